# baseline (speedup 1.0000x reference)
import functools

import jax
import jax.numpy as jnp
from jax import lax
from jax.experimental import pallas as pl
from jax.experimental.pallas import tpu as pltpu
from jax.experimental.pallas import tpu_sc as plsc

_N_EXPERTS = 64
_K = 8
_N_TOK = 32768
_NW = 32
_TPW = _N_TOK // _NW
_CHUNK = 256
_NCH = _TPW // _CHUNK
_NGRP = _CHUNK // 16


def _tc_block(x_ref, w_ref, b_ref, probs_ref, lt_ref):
    x = x_ref[...]
    w = w_ref[...]
    b_col = b_ref[:, 0:1]
    logits_t = jax.lax.dot_general(
        w, x,
        dimension_numbers=(((1,), (1,)), ((), ())),
        preferred_element_type=jnp.float32,
    ) + b_col
    lt_ref[...] = logits_t

    m = jnp.max(logits_t, axis=0, keepdims=True)
    e = jnp.exp(logits_t - m)
    s = jnp.sum(e, axis=0, keepdims=True)
    probs_ref[...] = (e / s).T


def _sc_mask_body(lt_hbm, mk_hbm, lt_v, mk_v):
    c = lax.axis_index("c")
    s = lax.axis_index("s")
    wid = s * 2 + c
    base = wid * _TPW
    one16 = jnp.ones((16,), jnp.int32)
    zero16 = jnp.zeros((16,), jnp.int32)
    neginf = jnp.full((16,), -jnp.inf, jnp.float32)

    def chunk_body(ch, carry):
        cb = base + ch * _CHUNK
        pltpu.sync_copy(lt_hbm.at[:, pl.ds(cb, _CHUNK)], lt_v)

        def group_body(g, carry2):
            col = g * 16
            v = [lt_v[e, pl.ds(col, 16)] for e in range(_N_EXPERTS)]
            mask_lo = zero16
            mask_hi = zero16
            for _ in range(_K):
                mx = v[0]
                for e in range(1, _N_EXPERTS):
                    mx = jnp.maximum(mx, v[e])
                chosen = jnp.full((16,), _N_EXPERTS, jnp.int32)
                for e in range(_N_EXPERTS - 1, -1, -1):
                    chosen = jnp.where(v[e] == mx, e, chosen)
                for e in range(_N_EXPERTS):
                    v[e] = jnp.where(chosen == e, neginf, v[e])
                bit = jnp.left_shift(one16, jnp.bitwise_and(chosen, 31))
                is_lo = chosen < 32
                mask_lo = mask_lo | jnp.where(is_lo, bit, zero16)
                mask_hi = mask_hi | jnp.where(is_lo, zero16, bit)
            mk_v[0, pl.ds(col, 16)] = mask_lo
            mk_v[1, pl.ds(col, 16)] = mask_hi
            return carry2

        lax.fori_loop(0, _NGRP, group_body, 0)
        pltpu.sync_copy(mk_v, mk_hbm.at[:, pl.ds(cb, _CHUNK)])
        return carry

    lax.fori_loop(0, _NCH, chunk_body, 0)


_sc_mask = functools.partial(
    pl.kernel,
    mesh=plsc.VectorSubcoreMesh(core_axis_name="c", subcore_axis_name="s"),
    out_type=jax.ShapeDtypeStruct((2, _N_TOK), jnp.int32),
    scratch_types=[
        pltpu.VMEM((_N_EXPERTS, _CHUNK), jnp.float32),
        pltpu.VMEM((2, _CHUNK), jnp.int32),
    ],
)(_sc_mask_body)


def _expand_block(p_ref, mask_ref):
    p = p_ref[...]
    block_t = p.shape[1]
    lo = jnp.broadcast_to(p[0:1, :], (_N_EXPERTS, block_t))
    hi = jnp.broadcast_to(p[1:2, :], (_N_EXPERTS, block_t))
    e_iota = lax.broadcasted_iota(jnp.int32, (_N_EXPERTS, block_t), 0)
    word = jnp.where(e_iota < 32, lo, hi)
    bits = jnp.bitwise_and(
        jax.lax.shift_right_logical(word, jnp.bitwise_and(e_iota, 31)), 1)
    mask_ref[...] = bits.astype(jnp.float32).T > 0.5


@jax.jit
def kernel(x, W, b):
    n_tokens, d_model = x.shape
    block_t = 4096
    grid = (n_tokens // block_t,)
    b2d = jnp.broadcast_to(b[:, None], (_N_EXPERTS, 128))

    probs, logits_t = pl.pallas_call(
        _tc_block,
        grid=grid,
        in_specs=[
            pl.BlockSpec((block_t, d_model), lambda i: (i, 0)),
            pl.BlockSpec((_N_EXPERTS, d_model), lambda i: (0, 0)),
            pl.BlockSpec((_N_EXPERTS, 128), lambda i: (0, 0)),
        ],
        out_specs=[
            pl.BlockSpec((block_t, _N_EXPERTS), lambda i: (i, 0)),
            pl.BlockSpec((_N_EXPERTS, block_t), lambda i: (0, i)),
        ],
        out_shape=[
            jax.ShapeDtypeStruct((n_tokens, _N_EXPERTS), jnp.float32),
            jax.ShapeDtypeStruct((_N_EXPERTS, n_tokens), jnp.float32),
        ],
    )(x, W, b2d)

    packed = _sc_mask(logits_t)

    mask = pl.pallas_call(
        _expand_block,
        grid=grid,
        in_specs=[pl.BlockSpec((2, block_t), lambda i: (0, i))],
        out_specs=pl.BlockSpec((block_t, _N_EXPERTS), lambda i: (i, 0)),
        out_shape=jax.ShapeDtypeStruct((n_tokens, _N_EXPERTS), jnp.bool_),
    )(packed)

    return (probs, mask)


# speedup vs baseline: 1.3069x; 1.3069x over previous
import jax
import jax.numpy as jnp
from jax.experimental import pallas as pl
from jax.experimental.pallas import tpu as pltpu

_N_EXPERTS = 64
_K = 8


def _router_block(x_ref, w_ref, b_ref, probs_ref, mask_ref, acc_ref):
    k = pl.program_id(1)
    x = x_ref[...]
    w = w_ref[...]
    partial = jax.lax.dot_general(
        w, x,
        dimension_numbers=(((1,), (1,)), ((), ())),
        preferred_element_type=jnp.float32,
    )

    @pl.when(k == 0)
    def _():
        acc_ref[...] = partial

    @pl.when(k == 1)
    def _():
        b_col = b_ref[:, 0:1]
        logits_t = acc_ref[...] + partial + b_col

        m = jnp.max(logits_t, axis=0, keepdims=True)
        e = jnp.exp(logits_t - m)
        s = jnp.sum(e, axis=0, keepdims=True)
        probs_t = e / s

        iota = jax.lax.broadcasted_iota(jnp.int32, logits_t.shape, 0)
        work = logits_t
        mask_t = jnp.zeros(logits_t.shape, dtype=jnp.float32)
        for _ in range(_K):
            mx = jnp.max(work, axis=0, keepdims=True)
            cand = jnp.where(work == mx, iota, _N_EXPERTS)
            first = jnp.min(cand, axis=0, keepdims=True)
            sel = iota == first
            mask_t = jnp.where(sel, 1.0, mask_t)
            work = jnp.where(sel, -jnp.inf, work)

        probs_ref[...] = probs_t.T
        mask_ref[...] = mask_t.T > 0.5


@jax.jit
def kernel(x, W, b):
    n_tokens, d_model = x.shape
    block_t = 8192
    block_d = d_model // 2
    grid = (n_tokens // block_t, 2)
    b2d = jnp.broadcast_to(b[:, None], (_N_EXPERTS, 128))

    probs, mask = pl.pallas_call(
        _router_block,
        grid=grid,
        in_specs=[
            pl.BlockSpec((block_t, block_d), lambda i, k: (i, k)),
            pl.BlockSpec((_N_EXPERTS, block_d), lambda i, k: (0, k)),
            pl.BlockSpec((_N_EXPERTS, 128), lambda i, k: (0, 0)),
        ],
        out_specs=[
            pl.BlockSpec((block_t, _N_EXPERTS), lambda i, k: (i, 0)),
            pl.BlockSpec((block_t, _N_EXPERTS), lambda i, k: (i, 0)),
        ],
        out_shape=[
            jax.ShapeDtypeStruct((n_tokens, _N_EXPERTS), jnp.float32),
            jax.ShapeDtypeStruct((n_tokens, _N_EXPERTS), jnp.bool_),
        ],
        scratch_shapes=[pltpu.VMEM((_N_EXPERTS, block_t), jnp.float32)],
    )(x, W, b2d)
    return (probs, mask)


# speedup vs baseline: 1.6665x; 1.2752x over previous
import jax
import jax.numpy as jnp
from jax.experimental import pallas as pl

_N_EXPERTS = 64
_K = 8


def _router_block(x_ref, w_ref, b_ref, probs_ref, mask_ref):
    x = x_ref[...]
    w = w_ref[...]
    b_col = b_ref[:, 0:1]
    logits_t = jax.lax.dot_general(
        w, x,
        dimension_numbers=(((1,), (1,)), ((), ())),
        preferred_element_type=jnp.float32,
    ) + b_col

    m = jnp.max(logits_t, axis=0, keepdims=True)
    e = jnp.exp(logits_t - m)
    s = jnp.sum(e, axis=0, keepdims=True)
    probs_t = e / s

    iota = jax.lax.broadcasted_iota(jnp.int32, logits_t.shape, 0)
    work = logits_t
    mask_t = jnp.zeros(logits_t.shape, dtype=jnp.float32)
    for _ in range(_K):
        mx = jnp.max(work, axis=0, keepdims=True)
        cand = jnp.where(work == mx, iota, _N_EXPERTS)
        first = jnp.min(cand, axis=0, keepdims=True)
        sel = iota == first
        mask_t = jnp.where(sel, 1.0, mask_t)
        work = jnp.where(sel, -jnp.inf, work)

    probs_ref[...] = probs_t.T
    mask_ref[...] = mask_t.T > 0.5


@jax.jit
def kernel(x, W, b):
    n_tokens, d_model = x.shape
    block_t = 4096
    grid = (n_tokens // block_t,)
    b2d = jnp.broadcast_to(b[:, None], (_N_EXPERTS, 128))

    probs, mask = pl.pallas_call(
        _router_block,
        grid=grid,
        in_specs=[
            pl.BlockSpec((block_t, d_model), lambda i: (i, 0)),
            pl.BlockSpec((_N_EXPERTS, d_model), lambda i: (0, 0)),
            pl.BlockSpec((_N_EXPERTS, 128), lambda i: (0, 0)),
        ],
        out_specs=[
            pl.BlockSpec((block_t, _N_EXPERTS), lambda i: (i, 0)),
            pl.BlockSpec((block_t, _N_EXPERTS), lambda i: (i, 0)),
        ],
        out_shape=[
            jax.ShapeDtypeStruct((n_tokens, _N_EXPERTS), jnp.float32),
            jax.ShapeDtypeStruct((n_tokens, _N_EXPERTS), jnp.bool_),
        ],
    )(x, W, b2d)
    return (probs, mask)
